# baseline (speedup 1.0000x reference)
import functools

import jax
import jax.numpy as jnp
from jax import lax
from jax.experimental import pallas as pl
from jax.experimental.pallas import tpu as pltpu
from jax.experimental.pallas import tpu_sc as plsc

_N = 10000
_E = 320000
_D = 128
_NC = 2
_NS = 16
_NW = _NC * _NS
_EW = _E // _NW
_CH = 80
_NCHUNK = _EW // _CH
_RPT = 624
_RPT_LAST = _N - (_NS - 1) * _RPT
_BN = 1000
_EPS_SCALE = float(1.0 / (1.0 + 1e-5) ** 0.5)


def _sc_mesh():
    return plsc.VectorSubcoreMesh(
        core_axis_name="c", subcore_axis_name="s",
        num_cores=_NC, num_subcores=_NS)


def _sc_degrees(src3, dst3, zeros_n):
    @functools.partial(
        pl.kernel,
        out_type=tuple(jax.ShapeDtypeStruct((_N,), jnp.float32)
                       for _ in range(2 * _NC)),
        mesh=_sc_mesh(),
        scratch_types=[
            pltpu.VMEM((_NCHUNK, _CH), jnp.int32),
            pltpu.VMEM((_NCHUNK, _CH), jnp.int32),
            pltpu.VMEM((_CH,), jnp.float32),
            pltpu.VMEM_SHARED((_N,), jnp.float32),
            pltpu.VMEM_SHARED((_N,), jnp.float32),
            pltpu.SemaphoreType.DMA,
            pltpu.SemaphoreType.DMA,
        ],
    )
    def deg_kernel(src_hbm, dst_hbm, zeros_hbm, dout0_hbm, din0_hbm,
                   dout1_hbm, din1_hbm, sidx, didx, ones_v, acc_o, acc_i,
                   semo, semi):
        c = lax.axis_index("c")
        s = lax.axis_index("s")
        wid = s * _NC + c
        pltpu.sync_copy(src_hbm.at[wid], sidx)
        pltpu.sync_copy(dst_hbm.at[wid], didx)
        for j in range(_CH // 16):
            ones_v[pl.ds(j * 16, 16)] = jnp.full((16,), 1.0, jnp.float32)

        @pl.when(s == 0)
        def _():
            pltpu.sync_copy(zeros_hbm, acc_o)
            pltpu.sync_copy(zeros_hbm, acc_i)

        plsc.subcore_barrier()

        def body(i, carry):
            @pl.when(i > 0)
            def _():
                pltpu.make_async_copy(ones_v, acc_o.at[sidx.at[i - 1]],
                                      semo).wait()
                pltpu.make_async_copy(ones_v, acc_i.at[didx.at[i - 1]],
                                      semi).wait()

            pltpu.async_copy(ones_v, acc_o.at[sidx.at[i]], semo, add=True)
            pltpu.async_copy(ones_v, acc_i.at[didx.at[i]], semi, add=True)
            return carry

        lax.fori_loop(0, _NCHUNK, body, 0)
        pltpu.make_async_copy(ones_v, acc_o.at[sidx.at[_NCHUNK - 1]],
                              semo).wait()
        pltpu.make_async_copy(ones_v, acc_i.at[didx.at[_NCHUNK - 1]],
                              semi).wait()
        plsc.subcore_barrier()

        @pl.when((s == 0) & (c == 0))
        def _():
            pltpu.sync_copy(acc_o, dout0_hbm)
            pltpu.sync_copy(acc_i, din0_hbm)

        @pl.when((s == 0) & (c == 1))
        def _():
            pltpu.sync_copy(acc_o, dout1_hbm)
            pltpu.sync_copy(acc_i, din1_hbm)

    return deg_kernel(src3, dst3, zeros_n)


def _sc_scatter(xs, src1, dst3, zeros_rows):
    @functools.partial(
        pl.kernel,
        out_type=tuple(jax.ShapeDtypeStruct((_N, _D), jnp.float32)
                       for _ in range(_NC)),
        mesh=_sc_mesh(),
        scratch_types=[
            pltpu.VMEM((_EW,), jnp.int32),
            pltpu.VMEM((_NCHUNK, _CH), jnp.int32),
            pltpu.VMEM((_CH, _D), jnp.float32),
            pltpu.VMEM((_CH, _D), jnp.float32),
            pltpu.VMEM_SHARED((_N, _D), jnp.float32),
            pltpu.SemaphoreType.DMA,
            pltpu.SemaphoreType.DMA,
            pltpu.SemaphoreType.DMA,
            pltpu.SemaphoreType.DMA,
        ],
    )
    def scat_kernel(xs_hbm, src_hbm, dst_hbm, zeros_hbm, out0_hbm, out1_hbm,
                    sidx, didx, rows0, rows1, acc, gsem0, gsem1, ssem0, ssem1):
        c = lax.axis_index("c")
        s = lax.axis_index("s")
        wid = s * _NC + c
        off = pl.multiple_of(s * _RPT, 8)
        cp_s = pltpu.async_copy(src_hbm.at[wid], sidx, gsem0)
        cp_d = pltpu.async_copy(dst_hbm.at[wid], didx, gsem1)

        @pl.when(s < _NS - 1)
        def _():
            pltpu.async_copy(zeros_hbm.at[pl.ds(0, _RPT)],
                             acc.at[pl.ds(off, _RPT)], ssem0)
            pltpu.make_async_copy(zeros_hbm.at[pl.ds(0, _RPT)],
                                  acc.at[pl.ds(off, _RPT)], ssem0).wait()

        @pl.when(s == _NS - 1)
        def _():
            pltpu.async_copy(zeros_hbm, acc.at[pl.ds(off, _RPT_LAST)], ssem0)
            pltpu.make_async_copy(zeros_hbm, acc.at[pl.ds(off, _RPT_LAST)],
                                  ssem0).wait()

        cp_s.wait()
        cp_d.wait()
        plsc.subcore_barrier()

        def gidx(i):
            return sidx.at[pl.ds(i * _CH, _CH)]

        pltpu.async_copy(xs_hbm.at[gidx(0)], rows0, gsem0)

        def body(j, carry):
            i0 = 2 * j
            i1 = i0 + 1

            @pl.when(j > 0)
            def _():
                pltpu.make_async_copy(rows1, acc.at[didx.at[i0 - 1]],
                                      ssem1).wait()

            pltpu.async_copy(xs_hbm.at[gidx(i1)], rows1, gsem1)
            pltpu.make_async_copy(xs_hbm.at[gidx(i0)], rows0, gsem0).wait()
            pltpu.async_copy(rows0, acc.at[didx.at[i0]], ssem0, add=True)
            pltpu.make_async_copy(rows0, acc.at[didx.at[i0]], ssem0).wait()

            @pl.when(i0 + 2 < _NCHUNK)
            def _():
                pltpu.async_copy(xs_hbm.at[gidx(i0 + 2)], rows0, gsem0)

            pltpu.make_async_copy(xs_hbm.at[gidx(i1)], rows1, gsem1).wait()
            pltpu.async_copy(rows1, acc.at[didx.at[i1]], ssem1, add=True)
            return carry

        lax.fori_loop(0, _NCHUNK // 2, body, 0)
        pltpu.make_async_copy(rows1, acc.at[didx.at[_NCHUNK - 2]],
                              ssem1).wait()
        pltpu.make_async_copy(xs_hbm.at[gidx(_NCHUNK - 1)], rows0,
                              gsem0).wait()
        pltpu.sync_copy(rows0, acc.at[didx.at[_NCHUNK - 1]], add=True)
        plsc.subcore_barrier()

        def copy_out(out_hbm):
            @pl.when(s < _NS - 1)
            def _():
                pltpu.sync_copy(acc.at[pl.ds(off, _RPT)],
                                out_hbm.at[pl.ds(off, _RPT)])

            @pl.when(s == _NS - 1)
            def _():
                pltpu.sync_copy(acc.at[pl.ds(off, _RPT_LAST)],
                                out_hbm.at[pl.ds(off, _RPT_LAST)])

        @pl.when(c == 0)
        def _():
            copy_out(out0_hbm)

        @pl.when(c == 1)
        def _():
            copy_out(out1_hbm)

    return scat_kernel(xs, src1, dst3, zeros_rows)


def _embed_body(h_ref, w_ref, b_ref, d0_ref, d1_ref, x_ref, xs_ref):
    x = jnp.dot(h_ref[...], w_ref[...],
                preferred_element_type=jnp.float32) + b_ref[...]
    deg = d0_ref[...] + d1_ref[...]
    nsrc = lax.rsqrt(jnp.where(deg > 0.0, deg, 1.0))
    x_ref[...] = x
    xs_ref[...] = x * nsrc


def _tc_embed(h, w, b2, dout0, dout1):
    return pl.pallas_call(
        _embed_body,
        grid=(_N // _BN,),
        in_specs=[
            pl.BlockSpec((_BN, _D), lambda i: (i, 0)),
            pl.BlockSpec((_D, _D), lambda i: (0, 0)),
            pl.BlockSpec((1, _D), lambda i: (0, 0)),
            pl.BlockSpec((_BN, 1), lambda i: (i, 0)),
            pl.BlockSpec((_BN, 1), lambda i: (i, 0)),
        ],
        out_specs=[
            pl.BlockSpec((_BN, _D), lambda i: (i, 0)),
            pl.BlockSpec((_BN, _D), lambda i: (i, 0)),
        ],
        out_shape=[jax.ShapeDtypeStruct((_N, _D), jnp.float32)] * 2,
    )(h, w, b2, dout0, dout1)


def _layer_body(p0_ref, p1_ref, xin_ref, di0_ref, di1_ref, snn_ref, w_ref,
                b_ref, gm_ref, bt_ref, do0_ref, do1_ref, xo_ref, xso_ref):
    dsum = di0_ref[...] + di1_ref[...]
    ndst = lax.rsqrt(jnp.where(dsum > 0.0, dsum, 1.0))
    agg = (p0_ref[...] + p1_ref[...]) * ndst
    y = jnp.dot(agg, w_ref[...],
                preferred_element_type=jnp.float32) + b_ref[...]
    y = y * snn_ref[...]
    y = y * (gm_ref[...] * _EPS_SCALE) + bt_ref[...]
    y = jnp.maximum(y, 0.0)
    x = xin_ref[...] + y
    osum = do0_ref[...] + do1_ref[...]
    nsrc = lax.rsqrt(jnp.where(osum > 0.0, osum, 1.0))
    xo_ref[...] = x
    xso_ref[...] = x * nsrc


def _tc_layer(p0, p1, x, din0, din1, snn, w, b2, gm2, bt2, dout0, dout1):
    return pl.pallas_call(
        _layer_body,
        grid=(_N // _BN,),
        in_specs=[
            pl.BlockSpec((_BN, _D), lambda i: (i, 0)),
            pl.BlockSpec((_BN, _D), lambda i: (i, 0)),
            pl.BlockSpec((_BN, _D), lambda i: (i, 0)),
            pl.BlockSpec((_BN, 1), lambda i: (i, 0)),
            pl.BlockSpec((_BN, 1), lambda i: (i, 0)),
            pl.BlockSpec((_BN, 1), lambda i: (i, 0)),
            pl.BlockSpec((_D, _D), lambda i: (0, 0)),
            pl.BlockSpec((1, _D), lambda i: (0, 0)),
            pl.BlockSpec((1, _D), lambda i: (0, 0)),
            pl.BlockSpec((1, _D), lambda i: (0, 0)),
            pl.BlockSpec((_BN, 1), lambda i: (i, 0)),
            pl.BlockSpec((_BN, 1), lambda i: (i, 0)),
        ],
        out_specs=[
            pl.BlockSpec((_BN, _D), lambda i: (i, 0)),
            pl.BlockSpec((_BN, _D), lambda i: (i, 0)),
        ],
        out_shape=[jax.ShapeDtypeStruct((_N, _D), jnp.float32)] * 2,
    )(p0, p1, x, din0, din1, snn, w, b2, gm2, bt2, dout0, dout1)


def _last_body(p0_ref, p1_ref, xin_ref, di0_ref, di1_ref, snn_ref, w_ref,
               b_ref, gm_ref, bt_ref, wm0_ref, bm0_ref, wm1_ref, bm1_ref,
               wm2_ref, bm2_ref, o_ref, acc_ref):
    dsum = di0_ref[...] + di1_ref[...]
    ndst = lax.rsqrt(jnp.where(dsum > 0.0, dsum, 1.0))
    agg = (p0_ref[...] + p1_ref[...]) * ndst
    y = jnp.dot(agg, w_ref[...],
                preferred_element_type=jnp.float32) + b_ref[...]
    y = y * snn_ref[...]
    y = y * (gm_ref[...] * _EPS_SCALE) + bt_ref[...]
    y = jnp.maximum(y, 0.0)
    x = xin_ref[...] + y
    i = pl.program_id(0)

    @pl.when(i == 0)
    def _():
        acc_ref[...] = jnp.zeros_like(acc_ref)

    acc_ref[...] += jnp.sum(x, axis=0, keepdims=True)

    @pl.when(i == pl.num_programs(0) - 1)
    def _():
        hg = acc_ref[...] * (1.0 / _N)
        z = jnp.dot(hg, wm0_ref[...], preferred_element_type=jnp.float32)
        z = jnp.maximum(z + bm0_ref[...], 0.0)
        z = jnp.dot(z, wm1_ref[...], preferred_element_type=jnp.float32)
        z = jnp.maximum(z + bm1_ref[...], 0.0)
        z = jnp.dot(z, wm2_ref[...], preferred_element_type=jnp.float32)
        o_ref[...] = z + bm2_ref[...]


def _tc_last(p0, p1, x, din0, din1, snn, w, b2, gm2, bt2,
             wm0, bm0, wm1, bm1, wm2, bm2):
    return pl.pallas_call(
        _last_body,
        grid=(_N // _BN,),
        in_specs=[
            pl.BlockSpec((_BN, _D), lambda i: (i, 0)),
            pl.BlockSpec((_BN, _D), lambda i: (i, 0)),
            pl.BlockSpec((_BN, _D), lambda i: (i, 0)),
            pl.BlockSpec((_BN, 1), lambda i: (i, 0)),
            pl.BlockSpec((_BN, 1), lambda i: (i, 0)),
            pl.BlockSpec((_BN, 1), lambda i: (i, 0)),
            pl.BlockSpec((_D, _D), lambda i: (0, 0)),
            pl.BlockSpec((1, _D), lambda i: (0, 0)),
            pl.BlockSpec((1, _D), lambda i: (0, 0)),
            pl.BlockSpec((1, _D), lambda i: (0, 0)),
            pl.BlockSpec(wm0.shape, lambda i: (0, 0)),
            pl.BlockSpec(bm0.shape, lambda i: (0, 0)),
            pl.BlockSpec(wm1.shape, lambda i: (0, 0)),
            pl.BlockSpec(bm1.shape, lambda i: (0, 0)),
            pl.BlockSpec(wm2.shape, lambda i: (0, 0)),
            pl.BlockSpec(bm2.shape, lambda i: (0, 0)),
        ],
        out_specs=pl.BlockSpec((1, 10), lambda i: (0, 0)),
        out_shape=jax.ShapeDtypeStruct((1, 10), jnp.float32),
        scratch_shapes=[pltpu.VMEM((1, _D), jnp.float32)],
    )(p0, p1, x, din0, din1, snn, w, b2, gm2, bt2,
      wm0, bm0, wm1, bm1, wm2, bm2)


def kernel(edge_index, h, e, snorm_n, snorm_e, W_emb, b_emb,
           W0, b0, gamma0, beta0, W1, b1, gamma1, beta1,
           W2, b2, gamma2, beta2, W3, b3, gamma3, beta3,
           Wm0, bm0, Wm1, bm1, Wm2, bm2):
    src1 = edge_index[0].reshape(_NW, _EW)
    src3 = edge_index[0].reshape(_NW, _NCHUNK, _CH)
    dst3 = edge_index[1].reshape(_NW, _NCHUNK, _CH)
    zeros_n = jnp.zeros((_N,), jnp.float32)
    zeros_rows = jnp.zeros((_RPT_LAST, _D), jnp.float32)

    dout0, din0, dout1, din1 = _sc_degrees(src3, dst3, zeros_n)
    dout0 = dout0.reshape(_N, 1)
    dout1 = dout1.reshape(_N, 1)
    din0 = din0.reshape(_N, 1)
    din1 = din1.reshape(_N, 1)
    x, xs = _tc_embed(h, W_emb, b_emb.reshape(1, _D), dout0, dout1)
    for w, b, gm, bt in ((W0, b0, gamma0, beta0), (W1, b1, gamma1, beta1),
                         (W2, b2, gamma2, beta2)):
        part0, part1 = _sc_scatter(xs, src1, dst3, zeros_rows)
        x, xs = _tc_layer(part0, part1, x, din0, din1, snorm_n,
                          w, b.reshape(1, _D), gm.reshape(1, _D),
                          bt.reshape(1, _D), dout0, dout1)
    part0, part1 = _sc_scatter(xs, src1, dst3, zeros_rows)
    return _tc_last(part0, part1, x, din0, din1, snorm_n,
                    W3, b3.reshape(1, _D), gamma3.reshape(1, _D),
                    beta3.reshape(1, _D), Wm0, bm0.reshape(1, -1),
                    Wm1, bm1.reshape(1, -1), Wm2, bm2.reshape(1, -1))


# speedup vs baseline: 1.0195x; 1.0195x over previous
import functools

import jax
import jax.numpy as jnp
from jax import lax
from jax.experimental import pallas as pl
from jax.experimental.pallas import tpu as pltpu
from jax.experimental.pallas import tpu_sc as plsc

_N = 10000
_E = 320000
_D = 128
_NC = 2
_NS = 16
_NW = _NC * _NS
_EW = _E // _NW
_CH = 80
_NCHUNK = _EW // _CH
_RPT = 624
_RPT_LAST = _N - (_NS - 1) * _RPT
_BN = 1000
_EPS_SCALE = float(1.0 / (1.0 + 1e-5) ** 0.5)


def _sc_mesh():
    return plsc.VectorSubcoreMesh(
        core_axis_name="c", subcore_axis_name="s",
        num_cores=_NC, num_subcores=_NS)


def _sc_degrees(src3, dst3, zeros_n):
    @functools.partial(
        pl.kernel,
        out_type=tuple(jax.ShapeDtypeStruct((_N,), jnp.float32)
                       for _ in range(2 * _NC)),
        mesh=_sc_mesh(),
        scratch_types=[
            pltpu.VMEM((_NCHUNK, _CH), jnp.int32),
            pltpu.VMEM((_NCHUNK, _CH), jnp.int32),
            pltpu.VMEM((_CH,), jnp.float32),
            pltpu.VMEM_SHARED((_N,), jnp.float32),
            pltpu.VMEM_SHARED((_N,), jnp.float32),
            pltpu.SemaphoreType.DMA,
            pltpu.SemaphoreType.DMA,
        ],
    )
    def deg_kernel(src_hbm, dst_hbm, zeros_hbm, dout0_hbm, din0_hbm,
                   dout1_hbm, din1_hbm, sidx, didx, ones_v, acc_o, acc_i,
                   semo, semi):
        c = lax.axis_index("c")
        s = lax.axis_index("s")
        wid = s * _NC + c
        pltpu.sync_copy(src_hbm.at[wid], sidx)
        pltpu.sync_copy(dst_hbm.at[wid], didx)
        for j in range(_CH // 16):
            ones_v[pl.ds(j * 16, 16)] = jnp.full((16,), 1.0, jnp.float32)

        @pl.when(s == 0)
        def _():
            pltpu.sync_copy(zeros_hbm, acc_o)
            pltpu.sync_copy(zeros_hbm, acc_i)

        plsc.subcore_barrier()

        def body(i, carry):
            @pl.when(i > 0)
            def _():
                pltpu.make_async_copy(ones_v, acc_o.at[sidx.at[i - 1]],
                                      semo).wait()
                pltpu.make_async_copy(ones_v, acc_i.at[didx.at[i - 1]],
                                      semi).wait()

            pltpu.async_copy(ones_v, acc_o.at[sidx.at[i]], semo, add=True)
            pltpu.async_copy(ones_v, acc_i.at[didx.at[i]], semi, add=True)
            return carry

        lax.fori_loop(0, _NCHUNK, body, 0)
        pltpu.make_async_copy(ones_v, acc_o.at[sidx.at[_NCHUNK - 1]],
                              semo).wait()
        pltpu.make_async_copy(ones_v, acc_i.at[didx.at[_NCHUNK - 1]],
                              semi).wait()
        plsc.subcore_barrier()

        @pl.when((s == 0) & (c == 0))
        def _():
            pltpu.sync_copy(acc_o, dout0_hbm)
            pltpu.sync_copy(acc_i, din0_hbm)

        @pl.when((s == 0) & (c == 1))
        def _():
            pltpu.sync_copy(acc_o, dout1_hbm)
            pltpu.sync_copy(acc_i, din1_hbm)

    return deg_kernel(src3, dst3, zeros_n)


def _sc_scatter(xs, src1, dst3, zeros_rows):
    @functools.partial(
        pl.kernel,
        out_type=tuple(jax.ShapeDtypeStruct((_N, _D), jnp.float32)
                       for _ in range(_NC)),
        mesh=_sc_mesh(),
        scratch_types=[
            pltpu.VMEM((_EW,), jnp.int32),
            pltpu.VMEM((_NCHUNK, _CH), jnp.int32),
            pltpu.VMEM((_CH, _D), jnp.float32),
            pltpu.VMEM((_CH, _D), jnp.float32),
            pltpu.VMEM_SHARED((_N, _D), jnp.float32),
            pltpu.SemaphoreType.DMA,
            pltpu.SemaphoreType.DMA,
            pltpu.SemaphoreType.DMA,
            pltpu.SemaphoreType.DMA,
        ],
    )
    def scat_kernel(xs_hbm, src_hbm, dst_hbm, zeros_hbm, out0_hbm, out1_hbm,
                    sidx, didx, rows0, rows1, acc, gsem0, gsem1, ssem0, ssem1):
        c = lax.axis_index("c")
        s = lax.axis_index("s")
        wid = s * _NC + c
        off = pl.multiple_of(s * _RPT, 8)
        cp_s = pltpu.async_copy(src_hbm.at[wid], sidx, gsem0)
        cp_d = pltpu.async_copy(dst_hbm.at[wid], didx, gsem1)

        @pl.when(s < _NS - 1)
        def _():
            pltpu.async_copy(zeros_hbm.at[pl.ds(0, _RPT)],
                             acc.at[pl.ds(off, _RPT)], ssem0)
            pltpu.make_async_copy(zeros_hbm.at[pl.ds(0, _RPT)],
                                  acc.at[pl.ds(off, _RPT)], ssem0).wait()

        @pl.when(s == _NS - 1)
        def _():
            pltpu.async_copy(zeros_hbm, acc.at[pl.ds(off, _RPT_LAST)], ssem0)
            pltpu.make_async_copy(zeros_hbm, acc.at[pl.ds(off, _RPT_LAST)],
                                  ssem0).wait()

        cp_s.wait()
        cp_d.wait()
        plsc.subcore_barrier()

        def gidx(i):
            return sidx.at[pl.ds(i * _CH, _CH)]

        pltpu.async_copy(xs_hbm.at[gidx(0)], rows0, gsem0)

        def body(j, carry):
            i0 = 2 * j
            i1 = i0 + 1

            @pl.when(j > 0)
            def _():
                pltpu.make_async_copy(rows1, acc.at[didx.at[i0 - 1]],
                                      ssem1).wait()

            pltpu.async_copy(xs_hbm.at[gidx(i1)], rows1, gsem1)
            pltpu.make_async_copy(xs_hbm.at[gidx(i0)], rows0, gsem0).wait()
            pltpu.async_copy(rows0, acc.at[didx.at[i0]], ssem0, add=True)
            pltpu.make_async_copy(rows0, acc.at[didx.at[i0]], ssem0).wait()

            @pl.when(i0 + 2 < _NCHUNK)
            def _():
                pltpu.async_copy(xs_hbm.at[gidx(i0 + 2)], rows0, gsem0)

            pltpu.make_async_copy(xs_hbm.at[gidx(i1)], rows1, gsem1).wait()
            pltpu.async_copy(rows1, acc.at[didx.at[i1]], ssem1, add=True)
            return carry

        lax.fori_loop(0, _NCHUNK // 2, body, 0)
        pltpu.make_async_copy(rows1, acc.at[didx.at[_NCHUNK - 2]],
                              ssem1).wait()
        pltpu.make_async_copy(xs_hbm.at[gidx(_NCHUNK - 1)], rows0,
                              gsem0).wait()
        pltpu.sync_copy(rows0, acc.at[didx.at[_NCHUNK - 1]], add=True)
        plsc.subcore_barrier()

        def copy_out(out_hbm):
            @pl.when(s < _NS - 1)
            def _():
                pltpu.sync_copy(acc.at[pl.ds(off, _RPT)],
                                out_hbm.at[pl.ds(off, _RPT)])

            @pl.when(s == _NS - 1)
            def _():
                pltpu.sync_copy(acc.at[pl.ds(off, _RPT_LAST)],
                                out_hbm.at[pl.ds(off, _RPT_LAST)])

        @pl.when(c == 0)
        def _():
            copy_out(out0_hbm)

        @pl.when(c == 1)
        def _():
            copy_out(out1_hbm)

    return scat_kernel(xs, src1, dst3, zeros_rows)


def _embed_body(h_ref, w_ref, b_ref, dout_ref, x_ref, xs_ref):
    x = jnp.dot(h_ref[...], w_ref[...],
                preferred_element_type=jnp.float32) + b_ref[...]
    deg = dout_ref[:, 0] + dout_ref[:, 1]
    nsrc = lax.rsqrt(jnp.where(deg > 0.0, deg, 1.0))
    x_ref[...] = x
    xs_ref[...] = x * nsrc[:, None]


def _tc_embed(h, w, b2, dout):
    return pl.pallas_call(
        _embed_body,
        grid=(_N // _BN,),
        in_specs=[
            pl.BlockSpec((_BN, _D), lambda i: (i, 0)),
            pl.BlockSpec((_D, _D), lambda i: (0, 0)),
            pl.BlockSpec((1, _D), lambda i: (0, 0)),
            pl.BlockSpec((_BN, _NC), lambda i: (i, 0)),
        ],
        out_specs=[
            pl.BlockSpec((_BN, _D), lambda i: (i, 0)),
            pl.BlockSpec((_BN, _D), lambda i: (i, 0)),
        ],
        out_shape=[jax.ShapeDtypeStruct((_N, _D), jnp.float32)] * 2,
    )(h, w, b2, dout)


def _layer_body(p0_ref, p1_ref, xin_ref, din_ref, snn_ref, w_ref,
                b_ref, gm_ref, bt_ref, dout_ref, xo_ref, xso_ref):
    dsum = din_ref[:, 0] + din_ref[:, 1]
    ndst = lax.rsqrt(jnp.where(dsum > 0.0, dsum, 1.0))
    agg = (p0_ref[...] + p1_ref[...]) * ndst[:, None]
    y = jnp.dot(agg, w_ref[...],
                preferred_element_type=jnp.float32) + b_ref[...]
    y = y * snn_ref[...]
    y = y * (gm_ref[...] * _EPS_SCALE) + bt_ref[...]
    y = jnp.maximum(y, 0.0)
    x = xin_ref[...] + y
    osum = dout_ref[:, 0] + dout_ref[:, 1]
    nsrc = lax.rsqrt(jnp.where(osum > 0.0, osum, 1.0))
    xo_ref[...] = x
    xso_ref[...] = x * nsrc[:, None]


def _tc_layer(p0, p1, x, din, snn, w, b2, gm2, bt2, dout):
    return pl.pallas_call(
        _layer_body,
        grid=(_N // _BN,),
        in_specs=[
            pl.BlockSpec((_BN, _D), lambda i: (i, 0)),
            pl.BlockSpec((_BN, _D), lambda i: (i, 0)),
            pl.BlockSpec((_BN, _D), lambda i: (i, 0)),
            pl.BlockSpec((_BN, _NC), lambda i: (i, 0)),
            pl.BlockSpec((_BN, 1), lambda i: (i, 0)),
            pl.BlockSpec((_D, _D), lambda i: (0, 0)),
            pl.BlockSpec((1, _D), lambda i: (0, 0)),
            pl.BlockSpec((1, _D), lambda i: (0, 0)),
            pl.BlockSpec((1, _D), lambda i: (0, 0)),
            pl.BlockSpec((_BN, _NC), lambda i: (i, 0)),
        ],
        out_specs=[
            pl.BlockSpec((_BN, _D), lambda i: (i, 0)),
            pl.BlockSpec((_BN, _D), lambda i: (i, 0)),
        ],
        out_shape=[jax.ShapeDtypeStruct((_N, _D), jnp.float32)] * 2,
    )(p0, p1, x, din, snn, w, b2, gm2, bt2, dout)


def _last_body(p0_ref, p1_ref, xin_ref, din_ref, snn_ref, w_ref,
               b_ref, gm_ref, bt_ref, wm0_ref, bm0_ref, wm1_ref, bm1_ref,
               wm2_ref, bm2_ref, o_ref, acc_ref):
    dsum = din_ref[:, 0] + din_ref[:, 1]
    ndst = lax.rsqrt(jnp.where(dsum > 0.0, dsum, 1.0))
    agg = (p0_ref[...] + p1_ref[...]) * ndst[:, None]
    y = jnp.dot(agg, w_ref[...],
                preferred_element_type=jnp.float32) + b_ref[...]
    y = y * snn_ref[...]
    y = y * (gm_ref[...] * _EPS_SCALE) + bt_ref[...]
    y = jnp.maximum(y, 0.0)
    x = xin_ref[...] + y
    i = pl.program_id(0)

    @pl.when(i == 0)
    def _():
        acc_ref[...] = jnp.zeros_like(acc_ref)

    acc_ref[...] += jnp.sum(x, axis=0, keepdims=True)

    @pl.when(i == pl.num_programs(0) - 1)
    def _():
        hg = acc_ref[...] * (1.0 / _N)
        z = jnp.dot(hg, wm0_ref[...], preferred_element_type=jnp.float32)
        z = jnp.maximum(z + bm0_ref[...], 0.0)
        z = jnp.dot(z, wm1_ref[...], preferred_element_type=jnp.float32)
        z = jnp.maximum(z + bm1_ref[...], 0.0)
        z = jnp.dot(z, wm2_ref[...], preferred_element_type=jnp.float32)
        o_ref[...] = z + bm2_ref[...]


def _tc_last(p0, p1, x, din, snn, w, b2, gm2, bt2,
             wm0, bm0, wm1, bm1, wm2, bm2):
    return pl.pallas_call(
        _last_body,
        grid=(_N // _BN,),
        in_specs=[
            pl.BlockSpec((_BN, _D), lambda i: (i, 0)),
            pl.BlockSpec((_BN, _D), lambda i: (i, 0)),
            pl.BlockSpec((_BN, _D), lambda i: (i, 0)),
            pl.BlockSpec((_BN, _NC), lambda i: (i, 0)),
            pl.BlockSpec((_BN, 1), lambda i: (i, 0)),
            pl.BlockSpec((_D, _D), lambda i: (0, 0)),
            pl.BlockSpec((1, _D), lambda i: (0, 0)),
            pl.BlockSpec((1, _D), lambda i: (0, 0)),
            pl.BlockSpec((1, _D), lambda i: (0, 0)),
            pl.BlockSpec(wm0.shape, lambda i: (0, 0)),
            pl.BlockSpec(bm0.shape, lambda i: (0, 0)),
            pl.BlockSpec(wm1.shape, lambda i: (0, 0)),
            pl.BlockSpec(bm1.shape, lambda i: (0, 0)),
            pl.BlockSpec(wm2.shape, lambda i: (0, 0)),
            pl.BlockSpec(bm2.shape, lambda i: (0, 0)),
        ],
        out_specs=pl.BlockSpec((1, 10), lambda i: (0, 0)),
        out_shape=jax.ShapeDtypeStruct((1, 10), jnp.float32),
        scratch_shapes=[pltpu.VMEM((1, _D), jnp.float32)],
    )(p0, p1, x, din, snn, w, b2, gm2, bt2,
      wm0, bm0, wm1, bm1, wm2, bm2)


def kernel(edge_index, h, e, snorm_n, snorm_e, W_emb, b_emb,
           W0, b0, gamma0, beta0, W1, b1, gamma1, beta1,
           W2, b2, gamma2, beta2, W3, b3, gamma3, beta3,
           Wm0, bm0, Wm1, bm1, Wm2, bm2):
    src1 = edge_index[0].reshape(_NW, _EW)
    src3 = edge_index[0].reshape(_NW, _NCHUNK, _CH)
    dst3 = edge_index[1].reshape(_NW, _NCHUNK, _CH)
    zeros_n = jnp.zeros((_N,), jnp.float32)
    zeros_rows = jnp.zeros((_RPT_LAST, _D), jnp.float32)

    dout0, din0, dout1, din1 = _sc_degrees(src3, dst3, zeros_n)
    dout = jnp.stack([dout0, dout1], axis=1)
    din = jnp.stack([din0, din1], axis=1)
    x, xs = _tc_embed(h, W_emb, b_emb.reshape(1, _D), dout)
    for w, b, gm, bt in ((W0, b0, gamma0, beta0), (W1, b1, gamma1, beta1),
                         (W2, b2, gamma2, beta2)):
        part0, part1 = _sc_scatter(xs, src1, dst3, zeros_rows)
        x, xs = _tc_layer(part0, part1, x, din, snorm_n,
                          w, b.reshape(1, _D), gm.reshape(1, _D),
                          bt.reshape(1, _D), dout)
    part0, part1 = _sc_scatter(xs, src1, dst3, zeros_rows)
    return _tc_last(part0, part1, x, din, snorm_n,
                    W3, b3.reshape(1, _D), gamma3.reshape(1, _D),
                    beta3.reshape(1, _D), Wm0, bm0.reshape(1, -1),
                    Wm1, bm1.reshape(1, -1), Wm2, bm2.reshape(1, -1))


# speedup vs baseline: 1.0310x; 1.0113x over previous
import functools

import jax
import jax.numpy as jnp
from jax import lax
from jax.experimental import pallas as pl
from jax.experimental.pallas import tpu as pltpu
from jax.experimental.pallas import tpu_sc as plsc

_N = 10000
_E = 320000
_D = 128
_NC = 2
_NS = 16
_NW = _NC * _NS
_EW = _E // _NW
_CH = 80
_NCHUNK = _EW // _CH
_RPT = 624
_RPT_LAST = _N - (_NS - 1) * _RPT
_BN = 2000
_EPS_SCALE = float(1.0 / (1.0 + 1e-5) ** 0.5)


def _sc_mesh():
    return plsc.VectorSubcoreMesh(
        core_axis_name="c", subcore_axis_name="s",
        num_cores=_NC, num_subcores=_NS)


def _sc_degrees(src3, dst3, zeros_n):
    @functools.partial(
        pl.kernel,
        out_type=tuple(jax.ShapeDtypeStruct((_N,), jnp.float32)
                       for _ in range(2 * _NC)),
        mesh=_sc_mesh(),
        scratch_types=[
            pltpu.VMEM((_NCHUNK, _CH), jnp.int32),
            pltpu.VMEM((_NCHUNK, _CH), jnp.int32),
            pltpu.VMEM((_CH,), jnp.float32),
            pltpu.VMEM_SHARED((_N,), jnp.float32),
            pltpu.VMEM_SHARED((_N,), jnp.float32),
            pltpu.SemaphoreType.DMA,
            pltpu.SemaphoreType.DMA,
        ],
    )
    def deg_kernel(src_hbm, dst_hbm, zeros_hbm, dout0_hbm, din0_hbm,
                   dout1_hbm, din1_hbm, sidx, didx, ones_v, acc_o, acc_i,
                   semo, semi):
        c = lax.axis_index("c")
        s = lax.axis_index("s")
        wid = s * _NC + c
        pltpu.sync_copy(src_hbm.at[wid], sidx)
        pltpu.sync_copy(dst_hbm.at[wid], didx)
        for j in range(_CH // 16):
            ones_v[pl.ds(j * 16, 16)] = jnp.full((16,), 1.0, jnp.float32)

        @pl.when(s == 0)
        def _():
            pltpu.sync_copy(zeros_hbm, acc_o)
            pltpu.sync_copy(zeros_hbm, acc_i)

        plsc.subcore_barrier()

        def body(i, carry):
            @pl.when(i > 0)
            def _():
                pltpu.make_async_copy(ones_v, acc_o.at[sidx.at[i - 1]],
                                      semo).wait()
                pltpu.make_async_copy(ones_v, acc_i.at[didx.at[i - 1]],
                                      semi).wait()

            pltpu.async_copy(ones_v, acc_o.at[sidx.at[i]], semo, add=True)
            pltpu.async_copy(ones_v, acc_i.at[didx.at[i]], semi, add=True)
            return carry

        lax.fori_loop(0, _NCHUNK, body, 0)
        pltpu.make_async_copy(ones_v, acc_o.at[sidx.at[_NCHUNK - 1]],
                              semo).wait()
        pltpu.make_async_copy(ones_v, acc_i.at[didx.at[_NCHUNK - 1]],
                              semi).wait()
        plsc.subcore_barrier()

        @pl.when((s == 0) & (c == 0))
        def _():
            pltpu.sync_copy(acc_o, dout0_hbm)
            pltpu.sync_copy(acc_i, din0_hbm)

        @pl.when((s == 0) & (c == 1))
        def _():
            pltpu.sync_copy(acc_o, dout1_hbm)
            pltpu.sync_copy(acc_i, din1_hbm)

    return deg_kernel(src3, dst3, zeros_n)


def _sc_scatter(xs, src1, dst3, zeros_rows):
    @functools.partial(
        pl.kernel,
        out_type=tuple(jax.ShapeDtypeStruct((_N, _D), jnp.float32)
                       for _ in range(_NC)),
        mesh=_sc_mesh(),
        scratch_types=[
            pltpu.VMEM((_EW,), jnp.int32),
            pltpu.VMEM((_NCHUNK, _CH), jnp.int32),
            pltpu.VMEM((_CH, _D), jnp.float32),
            pltpu.VMEM((_CH, _D), jnp.float32),
            pltpu.VMEM_SHARED((_N, _D), jnp.float32),
            pltpu.SemaphoreType.DMA,
            pltpu.SemaphoreType.DMA,
            pltpu.SemaphoreType.DMA,
            pltpu.SemaphoreType.DMA,
        ],
    )
    def scat_kernel(xs_hbm, src_hbm, dst_hbm, zeros_hbm, out0_hbm, out1_hbm,
                    sidx, didx, rows0, rows1, acc, gsem0, gsem1, ssem0, ssem1):
        c = lax.axis_index("c")
        s = lax.axis_index("s")
        wid = s * _NC + c
        off = pl.multiple_of(s * _RPT, 8)
        cp_s = pltpu.async_copy(src_hbm.at[wid], sidx, gsem0)
        cp_d = pltpu.async_copy(dst_hbm.at[wid], didx, gsem1)

        @pl.when(s < _NS - 1)
        def _():
            pltpu.async_copy(zeros_hbm.at[pl.ds(0, _RPT)],
                             acc.at[pl.ds(off, _RPT)], ssem0)
            pltpu.make_async_copy(zeros_hbm.at[pl.ds(0, _RPT)],
                                  acc.at[pl.ds(off, _RPT)], ssem0).wait()

        @pl.when(s == _NS - 1)
        def _():
            pltpu.async_copy(zeros_hbm, acc.at[pl.ds(off, _RPT_LAST)], ssem0)
            pltpu.make_async_copy(zeros_hbm, acc.at[pl.ds(off, _RPT_LAST)],
                                  ssem0).wait()

        cp_s.wait()
        cp_d.wait()
        plsc.subcore_barrier()

        def gidx(i):
            return sidx.at[pl.ds(i * _CH, _CH)]

        pltpu.async_copy(xs_hbm.at[gidx(0)], rows0, gsem0)

        def body(j, carry):
            i0 = 2 * j
            i1 = i0 + 1

            @pl.when(j > 0)
            def _():
                pltpu.make_async_copy(rows1, acc.at[didx.at[i0 - 1]],
                                      ssem1).wait()

            pltpu.async_copy(xs_hbm.at[gidx(i1)], rows1, gsem1)
            pltpu.make_async_copy(xs_hbm.at[gidx(i0)], rows0, gsem0).wait()
            pltpu.async_copy(rows0, acc.at[didx.at[i0]], ssem0, add=True)
            pltpu.make_async_copy(rows0, acc.at[didx.at[i0]], ssem0).wait()

            @pl.when(i0 + 2 < _NCHUNK)
            def _():
                pltpu.async_copy(xs_hbm.at[gidx(i0 + 2)], rows0, gsem0)

            pltpu.make_async_copy(xs_hbm.at[gidx(i1)], rows1, gsem1).wait()
            pltpu.async_copy(rows1, acc.at[didx.at[i1]], ssem1, add=True)
            return carry

        lax.fori_loop(0, _NCHUNK // 2, body, 0)
        pltpu.make_async_copy(rows1, acc.at[didx.at[_NCHUNK - 2]],
                              ssem1).wait()
        pltpu.make_async_copy(xs_hbm.at[gidx(_NCHUNK - 1)], rows0,
                              gsem0).wait()
        pltpu.sync_copy(rows0, acc.at[didx.at[_NCHUNK - 1]], add=True)
        plsc.subcore_barrier()

        def copy_out(out_hbm):
            @pl.when(s < _NS - 1)
            def _():
                pltpu.sync_copy(acc.at[pl.ds(off, _RPT)],
                                out_hbm.at[pl.ds(off, _RPT)])

            @pl.when(s == _NS - 1)
            def _():
                pltpu.sync_copy(acc.at[pl.ds(off, _RPT_LAST)],
                                out_hbm.at[pl.ds(off, _RPT_LAST)])

        @pl.when(c == 0)
        def _():
            copy_out(out0_hbm)

        @pl.when(c == 1)
        def _():
            copy_out(out1_hbm)

    return scat_kernel(xs, src1, dst3, zeros_rows)


def _embed_body(h_ref, w_ref, b_ref, dout_ref, x_ref, xs_ref):
    x = jnp.dot(h_ref[...], w_ref[...],
                preferred_element_type=jnp.float32) + b_ref[...]
    deg = dout_ref[:, 0] + dout_ref[:, 1]
    nsrc = lax.rsqrt(jnp.where(deg > 0.0, deg, 1.0))
    x_ref[...] = x
    xs_ref[...] = x * nsrc[:, None]


def _tc_embed(h, w, b2, dout):
    return pl.pallas_call(
        _embed_body,
        grid=(_N // _BN,),
        in_specs=[
            pl.BlockSpec((_BN, _D), lambda i: (i, 0)),
            pl.BlockSpec((_D, _D), lambda i: (0, 0)),
            pl.BlockSpec((1, _D), lambda i: (0, 0)),
            pl.BlockSpec((_BN, _NC), lambda i: (i, 0)),
        ],
        out_specs=[
            pl.BlockSpec((_BN, _D), lambda i: (i, 0)),
            pl.BlockSpec((_BN, _D), lambda i: (i, 0)),
        ],
        out_shape=[jax.ShapeDtypeStruct((_N, _D), jnp.float32)] * 2,
    )(h, w, b2, dout)


def _layer_body(p0_ref, p1_ref, xin_ref, din_ref, snn_ref, w_ref,
                b_ref, gm_ref, bt_ref, dout_ref, xo_ref, xso_ref):
    dsum = din_ref[:, 0] + din_ref[:, 1]
    ndst = lax.rsqrt(jnp.where(dsum > 0.0, dsum, 1.0))
    agg = (p0_ref[...] + p1_ref[...]) * ndst[:, None]
    y = jnp.dot(agg, w_ref[...],
                preferred_element_type=jnp.float32) + b_ref[...]
    y = y * snn_ref[...]
    y = y * (gm_ref[...] * _EPS_SCALE) + bt_ref[...]
    y = jnp.maximum(y, 0.0)
    x = xin_ref[...] + y
    osum = dout_ref[:, 0] + dout_ref[:, 1]
    nsrc = lax.rsqrt(jnp.where(osum > 0.0, osum, 1.0))
    xo_ref[...] = x
    xso_ref[...] = x * nsrc[:, None]


def _tc_layer(p0, p1, x, din, snn, w, b2, gm2, bt2, dout):
    return pl.pallas_call(
        _layer_body,
        grid=(_N // _BN,),
        in_specs=[
            pl.BlockSpec((_BN, _D), lambda i: (i, 0)),
            pl.BlockSpec((_BN, _D), lambda i: (i, 0)),
            pl.BlockSpec((_BN, _D), lambda i: (i, 0)),
            pl.BlockSpec((_BN, _NC), lambda i: (i, 0)),
            pl.BlockSpec((_BN, 1), lambda i: (i, 0)),
            pl.BlockSpec((_D, _D), lambda i: (0, 0)),
            pl.BlockSpec((1, _D), lambda i: (0, 0)),
            pl.BlockSpec((1, _D), lambda i: (0, 0)),
            pl.BlockSpec((1, _D), lambda i: (0, 0)),
            pl.BlockSpec((_BN, _NC), lambda i: (i, 0)),
        ],
        out_specs=[
            pl.BlockSpec((_BN, _D), lambda i: (i, 0)),
            pl.BlockSpec((_BN, _D), lambda i: (i, 0)),
        ],
        out_shape=[jax.ShapeDtypeStruct((_N, _D), jnp.float32)] * 2,
    )(p0, p1, x, din, snn, w, b2, gm2, bt2, dout)


def _last_body(p0_ref, p1_ref, xin_ref, din_ref, snn_ref, w_ref,
               b_ref, gm_ref, bt_ref, wm0_ref, bm0_ref, wm1_ref, bm1_ref,
               wm2_ref, bm2_ref, o_ref, acc_ref):
    dsum = din_ref[:, 0] + din_ref[:, 1]
    ndst = lax.rsqrt(jnp.where(dsum > 0.0, dsum, 1.0))
    agg = (p0_ref[...] + p1_ref[...]) * ndst[:, None]
    y = jnp.dot(agg, w_ref[...],
                preferred_element_type=jnp.float32) + b_ref[...]
    y = y * snn_ref[...]
    y = y * (gm_ref[...] * _EPS_SCALE) + bt_ref[...]
    y = jnp.maximum(y, 0.0)
    x = xin_ref[...] + y
    i = pl.program_id(0)

    @pl.when(i == 0)
    def _():
        acc_ref[...] = jnp.zeros_like(acc_ref)

    acc_ref[...] += jnp.sum(x, axis=0, keepdims=True)

    @pl.when(i == pl.num_programs(0) - 1)
    def _():
        hg = acc_ref[...] * (1.0 / _N)
        z = jnp.dot(hg, wm0_ref[...], preferred_element_type=jnp.float32)
        z = jnp.maximum(z + bm0_ref[...], 0.0)
        z = jnp.dot(z, wm1_ref[...], preferred_element_type=jnp.float32)
        z = jnp.maximum(z + bm1_ref[...], 0.0)
        z = jnp.dot(z, wm2_ref[...], preferred_element_type=jnp.float32)
        o_ref[...] = z + bm2_ref[...]


def _tc_last(p0, p1, x, din, snn, w, b2, gm2, bt2,
             wm0, bm0, wm1, bm1, wm2, bm2):
    return pl.pallas_call(
        _last_body,
        grid=(_N // _BN,),
        in_specs=[
            pl.BlockSpec((_BN, _D), lambda i: (i, 0)),
            pl.BlockSpec((_BN, _D), lambda i: (i, 0)),
            pl.BlockSpec((_BN, _D), lambda i: (i, 0)),
            pl.BlockSpec((_BN, _NC), lambda i: (i, 0)),
            pl.BlockSpec((_BN, 1), lambda i: (i, 0)),
            pl.BlockSpec((_D, _D), lambda i: (0, 0)),
            pl.BlockSpec((1, _D), lambda i: (0, 0)),
            pl.BlockSpec((1, _D), lambda i: (0, 0)),
            pl.BlockSpec((1, _D), lambda i: (0, 0)),
            pl.BlockSpec(wm0.shape, lambda i: (0, 0)),
            pl.BlockSpec(bm0.shape, lambda i: (0, 0)),
            pl.BlockSpec(wm1.shape, lambda i: (0, 0)),
            pl.BlockSpec(bm1.shape, lambda i: (0, 0)),
            pl.BlockSpec(wm2.shape, lambda i: (0, 0)),
            pl.BlockSpec(bm2.shape, lambda i: (0, 0)),
        ],
        out_specs=pl.BlockSpec((1, 10), lambda i: (0, 0)),
        out_shape=jax.ShapeDtypeStruct((1, 10), jnp.float32),
        scratch_shapes=[pltpu.VMEM((1, _D), jnp.float32)],
    )(p0, p1, x, din, snn, w, b2, gm2, bt2,
      wm0, bm0, wm1, bm1, wm2, bm2)


def kernel(edge_index, h, e, snorm_n, snorm_e, W_emb, b_emb,
           W0, b0, gamma0, beta0, W1, b1, gamma1, beta1,
           W2, b2, gamma2, beta2, W3, b3, gamma3, beta3,
           Wm0, bm0, Wm1, bm1, Wm2, bm2):
    src1 = edge_index[0].reshape(_NW, _EW)
    src3 = edge_index[0].reshape(_NW, _NCHUNK, _CH)
    dst3 = edge_index[1].reshape(_NW, _NCHUNK, _CH)
    zeros_n = jnp.zeros((_N,), jnp.float32)
    zeros_rows = jnp.zeros((_RPT_LAST, _D), jnp.float32)

    dout0, din0, dout1, din1 = _sc_degrees(src3, dst3, zeros_n)
    dout = jnp.stack([dout0, dout1], axis=1)
    din = jnp.stack([din0, din1], axis=1)
    x, xs = _tc_embed(h, W_emb, b_emb.reshape(1, _D), dout)
    for w, b, gm, bt in ((W0, b0, gamma0, beta0), (W1, b1, gamma1, beta1),
                         (W2, b2, gamma2, beta2)):
        part0, part1 = _sc_scatter(xs, src1, dst3, zeros_rows)
        x, xs = _tc_layer(part0, part1, x, din, snorm_n,
                          w, b.reshape(1, _D), gm.reshape(1, _D),
                          bt.reshape(1, _D), dout)
    part0, part1 = _sc_scatter(xs, src1, dst3, zeros_rows)
    return _tc_last(part0, part1, x, din, snorm_n,
                    W3, b3.reshape(1, _D), gamma3.reshape(1, _D),
                    beta3.reshape(1, _D), Wm0, bm0.reshape(1, -1),
                    Wm1, bm1.reshape(1, -1), Wm2, bm2.reshape(1, -1))


# speedup vs baseline: 1.0414x; 1.0101x over previous
import functools

import jax
import jax.numpy as jnp
from jax import lax
from jax.experimental import pallas as pl
from jax.experimental.pallas import tpu as pltpu
from jax.experimental.pallas import tpu_sc as plsc

_N = 10000
_E = 320000
_D = 128
_NC = 2
_NS = 16
_NW = _NC * _NS
_EW = _E // _NW
_CH = 80
_NCHUNK = _EW // _CH
_RPT = 624
_RPT_LAST = _N - (_NS - 1) * _RPT
_BN = 5000
_EPS_SCALE = float(1.0 / (1.0 + 1e-5) ** 0.5)


def _sc_mesh():
    return plsc.VectorSubcoreMesh(
        core_axis_name="c", subcore_axis_name="s",
        num_cores=_NC, num_subcores=_NS)


def _sc_degrees(src3, dst3, zeros_n):
    @functools.partial(
        pl.kernel,
        out_type=tuple(jax.ShapeDtypeStruct((_N,), jnp.float32)
                       for _ in range(2 * _NC)),
        mesh=_sc_mesh(),
        scratch_types=[
            pltpu.VMEM((_NCHUNK, _CH), jnp.int32),
            pltpu.VMEM((_NCHUNK, _CH), jnp.int32),
            pltpu.VMEM((_CH,), jnp.float32),
            pltpu.VMEM_SHARED((_N,), jnp.float32),
            pltpu.VMEM_SHARED((_N,), jnp.float32),
            pltpu.SemaphoreType.DMA,
            pltpu.SemaphoreType.DMA,
        ],
    )
    def deg_kernel(src_hbm, dst_hbm, zeros_hbm, dout0_hbm, din0_hbm,
                   dout1_hbm, din1_hbm, sidx, didx, ones_v, acc_o, acc_i,
                   semo, semi):
        c = lax.axis_index("c")
        s = lax.axis_index("s")
        wid = s * _NC + c
        pltpu.sync_copy(src_hbm.at[wid], sidx)
        pltpu.sync_copy(dst_hbm.at[wid], didx)
        for j in range(_CH // 16):
            ones_v[pl.ds(j * 16, 16)] = jnp.full((16,), 1.0, jnp.float32)

        @pl.when(s == 0)
        def _():
            pltpu.sync_copy(zeros_hbm, acc_o)
            pltpu.sync_copy(zeros_hbm, acc_i)

        plsc.subcore_barrier()

        def body(i, carry):
            @pl.when(i > 0)
            def _():
                pltpu.make_async_copy(ones_v, acc_o.at[sidx.at[i - 1]],
                                      semo).wait()
                pltpu.make_async_copy(ones_v, acc_i.at[didx.at[i - 1]],
                                      semi).wait()

            pltpu.async_copy(ones_v, acc_o.at[sidx.at[i]], semo, add=True)
            pltpu.async_copy(ones_v, acc_i.at[didx.at[i]], semi, add=True)
            return carry

        lax.fori_loop(0, _NCHUNK, body, 0)
        pltpu.make_async_copy(ones_v, acc_o.at[sidx.at[_NCHUNK - 1]],
                              semo).wait()
        pltpu.make_async_copy(ones_v, acc_i.at[didx.at[_NCHUNK - 1]],
                              semi).wait()
        plsc.subcore_barrier()

        @pl.when((s == 0) & (c == 0))
        def _():
            pltpu.sync_copy(acc_o, dout0_hbm)
            pltpu.sync_copy(acc_i, din0_hbm)

        @pl.when((s == 0) & (c == 1))
        def _():
            pltpu.sync_copy(acc_o, dout1_hbm)
            pltpu.sync_copy(acc_i, din1_hbm)

    return deg_kernel(src3, dst3, zeros_n)


def _sc_scatter(xs, src1, dst3, zeros_rows):
    @functools.partial(
        pl.kernel,
        out_type=tuple(jax.ShapeDtypeStruct((_N, _D), jnp.float32)
                       for _ in range(_NC)),
        mesh=_sc_mesh(),
        scratch_types=[
            pltpu.VMEM((_EW,), jnp.int32),
            pltpu.VMEM((_NCHUNK, _CH), jnp.int32),
            pltpu.VMEM((_CH, _D), jnp.float32),
            pltpu.VMEM((_CH, _D), jnp.float32),
            pltpu.VMEM_SHARED((_N, _D), jnp.float32),
            pltpu.SemaphoreType.DMA,
            pltpu.SemaphoreType.DMA,
            pltpu.SemaphoreType.DMA,
            pltpu.SemaphoreType.DMA,
        ],
    )
    def scat_kernel(xs_hbm, src_hbm, dst_hbm, zeros_hbm, out0_hbm, out1_hbm,
                    sidx, didx, rows0, rows1, acc, gsem0, gsem1, ssem0, ssem1):
        c = lax.axis_index("c")
        s = lax.axis_index("s")
        wid = s * _NC + c
        off = pl.multiple_of(s * _RPT, 8)
        cp_s = pltpu.async_copy(src_hbm.at[wid], sidx, gsem0)
        cp_d = pltpu.async_copy(dst_hbm.at[wid], didx, gsem1)

        @pl.when(s < _NS - 1)
        def _():
            pltpu.async_copy(zeros_hbm.at[pl.ds(0, _RPT)],
                             acc.at[pl.ds(off, _RPT)], ssem0)
            pltpu.make_async_copy(zeros_hbm.at[pl.ds(0, _RPT)],
                                  acc.at[pl.ds(off, _RPT)], ssem0).wait()

        @pl.when(s == _NS - 1)
        def _():
            pltpu.async_copy(zeros_hbm, acc.at[pl.ds(off, _RPT_LAST)], ssem0)
            pltpu.make_async_copy(zeros_hbm, acc.at[pl.ds(off, _RPT_LAST)],
                                  ssem0).wait()

        cp_s.wait()
        cp_d.wait()
        plsc.subcore_barrier()

        def gidx(i):
            return sidx.at[pl.ds(i * _CH, _CH)]

        pltpu.async_copy(xs_hbm.at[gidx(0)], rows0, gsem0)

        def body(j, carry):
            i0 = 2 * j
            i1 = i0 + 1

            @pl.when(j > 0)
            def _():
                pltpu.make_async_copy(rows1, acc.at[didx.at[i0 - 1]],
                                      ssem1).wait()

            pltpu.async_copy(xs_hbm.at[gidx(i1)], rows1, gsem1)
            pltpu.make_async_copy(xs_hbm.at[gidx(i0)], rows0, gsem0).wait()
            pltpu.async_copy(rows0, acc.at[didx.at[i0]], ssem0, add=True)
            pltpu.make_async_copy(rows0, acc.at[didx.at[i0]], ssem0).wait()

            @pl.when(i0 + 2 < _NCHUNK)
            def _():
                pltpu.async_copy(xs_hbm.at[gidx(i0 + 2)], rows0, gsem0)

            pltpu.make_async_copy(xs_hbm.at[gidx(i1)], rows1, gsem1).wait()
            pltpu.async_copy(rows1, acc.at[didx.at[i1]], ssem1, add=True)
            return carry

        lax.fori_loop(0, _NCHUNK // 2, body, 0)
        pltpu.make_async_copy(rows1, acc.at[didx.at[_NCHUNK - 2]],
                              ssem1).wait()
        pltpu.make_async_copy(xs_hbm.at[gidx(_NCHUNK - 1)], rows0,
                              gsem0).wait()
        pltpu.sync_copy(rows0, acc.at[didx.at[_NCHUNK - 1]], add=True)
        plsc.subcore_barrier()

        def copy_out(out_hbm):
            @pl.when(s < _NS - 1)
            def _():
                pltpu.sync_copy(acc.at[pl.ds(off, _RPT)],
                                out_hbm.at[pl.ds(off, _RPT)])

            @pl.when(s == _NS - 1)
            def _():
                pltpu.sync_copy(acc.at[pl.ds(off, _RPT_LAST)],
                                out_hbm.at[pl.ds(off, _RPT_LAST)])

        @pl.when(c == 0)
        def _():
            copy_out(out0_hbm)

        @pl.when(c == 1)
        def _():
            copy_out(out1_hbm)

    return scat_kernel(xs, src1, dst3, zeros_rows)


def _embed_body(h_ref, w_ref, b_ref, dout_ref, x_ref, xs_ref):
    x = jnp.dot(h_ref[...], w_ref[...],
                preferred_element_type=jnp.float32) + b_ref[...]
    deg = dout_ref[:, 0] + dout_ref[:, 1]
    nsrc = lax.rsqrt(jnp.where(deg > 0.0, deg, 1.0))
    x_ref[...] = x
    xs_ref[...] = x * nsrc[:, None]


def _tc_embed(h, w, b2, dout):
    return pl.pallas_call(
        _embed_body,
        grid=(_N // _BN,),
        in_specs=[
            pl.BlockSpec((_BN, _D), lambda i: (i, 0)),
            pl.BlockSpec((_D, _D), lambda i: (0, 0)),
            pl.BlockSpec((1, _D), lambda i: (0, 0)),
            pl.BlockSpec((_BN, _NC), lambda i: (i, 0)),
        ],
        out_specs=[
            pl.BlockSpec((_BN, _D), lambda i: (i, 0)),
            pl.BlockSpec((_BN, _D), lambda i: (i, 0)),
        ],
        out_shape=[jax.ShapeDtypeStruct((_N, _D), jnp.float32)] * 2,
    )(h, w, b2, dout)


def _layer_body(p0_ref, p1_ref, xin_ref, din_ref, snn_ref, w_ref,
                b_ref, gm_ref, bt_ref, dout_ref, xo_ref, xso_ref):
    dsum = din_ref[:, 0] + din_ref[:, 1]
    ndst = lax.rsqrt(jnp.where(dsum > 0.0, dsum, 1.0))
    agg = (p0_ref[...] + p1_ref[...]) * ndst[:, None]
    y = jnp.dot(agg, w_ref[...],
                preferred_element_type=jnp.float32) + b_ref[...]
    y = y * snn_ref[...]
    y = y * (gm_ref[...] * _EPS_SCALE) + bt_ref[...]
    y = jnp.maximum(y, 0.0)
    x = xin_ref[...] + y
    osum = dout_ref[:, 0] + dout_ref[:, 1]
    nsrc = lax.rsqrt(jnp.where(osum > 0.0, osum, 1.0))
    xo_ref[...] = x
    xso_ref[...] = x * nsrc[:, None]


def _tc_layer(p0, p1, x, din, snn, w, b2, gm2, bt2, dout):
    return pl.pallas_call(
        _layer_body,
        grid=(_N // _BN,),
        in_specs=[
            pl.BlockSpec((_BN, _D), lambda i: (i, 0)),
            pl.BlockSpec((_BN, _D), lambda i: (i, 0)),
            pl.BlockSpec((_BN, _D), lambda i: (i, 0)),
            pl.BlockSpec((_BN, _NC), lambda i: (i, 0)),
            pl.BlockSpec((_BN, 1), lambda i: (i, 0)),
            pl.BlockSpec((_D, _D), lambda i: (0, 0)),
            pl.BlockSpec((1, _D), lambda i: (0, 0)),
            pl.BlockSpec((1, _D), lambda i: (0, 0)),
            pl.BlockSpec((1, _D), lambda i: (0, 0)),
            pl.BlockSpec((_BN, _NC), lambda i: (i, 0)),
        ],
        out_specs=[
            pl.BlockSpec((_BN, _D), lambda i: (i, 0)),
            pl.BlockSpec((_BN, _D), lambda i: (i, 0)),
        ],
        out_shape=[jax.ShapeDtypeStruct((_N, _D), jnp.float32)] * 2,
    )(p0, p1, x, din, snn, w, b2, gm2, bt2, dout)


def _last_body(p0_ref, p1_ref, xin_ref, din_ref, snn_ref, w_ref,
               b_ref, gm_ref, bt_ref, wm0_ref, bm0_ref, wm1_ref, bm1_ref,
               wm2_ref, bm2_ref, o_ref, acc_ref):
    dsum = din_ref[:, 0] + din_ref[:, 1]
    ndst = lax.rsqrt(jnp.where(dsum > 0.0, dsum, 1.0))
    agg = (p0_ref[...] + p1_ref[...]) * ndst[:, None]
    y = jnp.dot(agg, w_ref[...],
                preferred_element_type=jnp.float32) + b_ref[...]
    y = y * snn_ref[...]
    y = y * (gm_ref[...] * _EPS_SCALE) + bt_ref[...]
    y = jnp.maximum(y, 0.0)
    x = xin_ref[...] + y
    i = pl.program_id(0)

    @pl.when(i == 0)
    def _():
        acc_ref[...] = jnp.zeros_like(acc_ref)

    acc_ref[...] += jnp.sum(x, axis=0, keepdims=True)

    @pl.when(i == pl.num_programs(0) - 1)
    def _():
        hg = acc_ref[...] * (1.0 / _N)
        z = jnp.dot(hg, wm0_ref[...], preferred_element_type=jnp.float32)
        z = jnp.maximum(z + bm0_ref[...], 0.0)
        z = jnp.dot(z, wm1_ref[...], preferred_element_type=jnp.float32)
        z = jnp.maximum(z + bm1_ref[...], 0.0)
        z = jnp.dot(z, wm2_ref[...], preferred_element_type=jnp.float32)
        o_ref[...] = z + bm2_ref[...]


def _tc_last(p0, p1, x, din, snn, w, b2, gm2, bt2,
             wm0, bm0, wm1, bm1, wm2, bm2):
    return pl.pallas_call(
        _last_body,
        grid=(_N // _BN,),
        in_specs=[
            pl.BlockSpec((_BN, _D), lambda i: (i, 0)),
            pl.BlockSpec((_BN, _D), lambda i: (i, 0)),
            pl.BlockSpec((_BN, _D), lambda i: (i, 0)),
            pl.BlockSpec((_BN, _NC), lambda i: (i, 0)),
            pl.BlockSpec((_BN, 1), lambda i: (i, 0)),
            pl.BlockSpec((_D, _D), lambda i: (0, 0)),
            pl.BlockSpec((1, _D), lambda i: (0, 0)),
            pl.BlockSpec((1, _D), lambda i: (0, 0)),
            pl.BlockSpec((1, _D), lambda i: (0, 0)),
            pl.BlockSpec(wm0.shape, lambda i: (0, 0)),
            pl.BlockSpec(bm0.shape, lambda i: (0, 0)),
            pl.BlockSpec(wm1.shape, lambda i: (0, 0)),
            pl.BlockSpec(bm1.shape, lambda i: (0, 0)),
            pl.BlockSpec(wm2.shape, lambda i: (0, 0)),
            pl.BlockSpec(bm2.shape, lambda i: (0, 0)),
        ],
        out_specs=pl.BlockSpec((1, 10), lambda i: (0, 0)),
        out_shape=jax.ShapeDtypeStruct((1, 10), jnp.float32),
        scratch_shapes=[pltpu.VMEM((1, _D), jnp.float32)],
    )(p0, p1, x, din, snn, w, b2, gm2, bt2,
      wm0, bm0, wm1, bm1, wm2, bm2)


def kernel(edge_index, h, e, snorm_n, snorm_e, W_emb, b_emb,
           W0, b0, gamma0, beta0, W1, b1, gamma1, beta1,
           W2, b2, gamma2, beta2, W3, b3, gamma3, beta3,
           Wm0, bm0, Wm1, bm1, Wm2, bm2):
    src1 = edge_index[0].reshape(_NW, _EW)
    src3 = edge_index[0].reshape(_NW, _NCHUNK, _CH)
    dst3 = edge_index[1].reshape(_NW, _NCHUNK, _CH)
    zeros_n = jnp.zeros((_N,), jnp.float32)
    zeros_rows = jnp.zeros((_RPT_LAST, _D), jnp.float32)

    dout0, din0, dout1, din1 = _sc_degrees(src3, dst3, zeros_n)
    dout = jnp.stack([dout0, dout1], axis=1)
    din = jnp.stack([din0, din1], axis=1)
    x, xs = _tc_embed(h, W_emb, b_emb.reshape(1, _D), dout)
    for w, b, gm, bt in ((W0, b0, gamma0, beta0), (W1, b1, gamma1, beta1),
                         (W2, b2, gamma2, beta2)):
        part0, part1 = _sc_scatter(xs, src1, dst3, zeros_rows)
        x, xs = _tc_layer(part0, part1, x, din, snorm_n,
                          w, b.reshape(1, _D), gm.reshape(1, _D),
                          bt.reshape(1, _D), dout)
    part0, part1 = _sc_scatter(xs, src1, dst3, zeros_rows)
    return _tc_last(part0, part1, x, din, snorm_n,
                    W3, b3.reshape(1, _D), gamma3.reshape(1, _D),
                    beta3.reshape(1, _D), Wm0, bm0.reshape(1, -1),
                    Wm1, bm1.reshape(1, -1), Wm2, bm2.reshape(1, -1))
